# baseline (device time: 164019 ns/iter reference)
import jax
import jax.numpy as jnp
from jax import lax
from jax.experimental import pallas as pl
from jax.experimental.pallas import tpu as pltpu

B = 32
NB = 256
BS = 32
H = 16
D = 128
P_GLOBAL = 512
P_LOCAL = 256
KEYS = P_LOCAL * BS
NEG = -1e30
SCALE = D ** -0.5


def kernel(Q, K, V, bt, lens):
    my_x = lax.axis_index("x")

    slot_ok = jnp.arange(NB, dtype=jnp.int32)[None, :] < lens[:, None]
    hits = slot_ok[:, :, None] & (
        bt[:, :, None] == jnp.arange(P_GLOBAL, dtype=jnp.int32)[None, None, :]
    )
    W = jnp.sum(hits.astype(jnp.float32), axis=1)
    Wx = lax.dynamic_slice(W, (0, my_x * P_LOCAL), (B, P_LOCAL))
    logw = jnp.where(Wx > 0, jnp.log(jnp.maximum(Wx, 1e-37)), NEG)
    logw_keys = jnp.repeat(logw, BS, axis=1)

    q2 = Q.reshape(B, H * D)
    k2 = K.reshape(KEYS, H * D)
    v2 = V.reshape(KEYS, H * D)

    def body(q_ref, k_ref, v_ref, lw_ref, out_ref,
             o_acc, ml_acc, o_rcv, ml_rcv, send_sems, recv_sems):
        h = pl.program_id(0)
        q = q_ref[...].astype(jnp.bfloat16)
        k = k_ref[...].astype(jnp.bfloat16)
        v = v_ref[...].astype(jnp.bfloat16)
        s = lax.dot_general(
            q, k, (((1,), (1,)), ((), ())),
            preferred_element_type=jnp.float32,
        )
        s = s * SCALE + lw_ref[...]
        m = jnp.max(s, axis=1, keepdims=True)
        p = jnp.where(m > -1e29, jnp.exp(s - m), 0.0)
        l = jnp.sum(p, axis=1, keepdims=True)
        o = lax.dot_general(
            p.astype(jnp.bfloat16), v, (((1,), (0,)), ((), ())),
            preferred_element_type=jnp.float32,
        )
        o_acc[h] = o
        ml_acc[0, h] = m
        ml_acc[1, h] = l

        @pl.when(h == H - 1)
        def _():
            peer = (1 - lax.axis_index("x"), lax.axis_index("y"))
            rdma_o = pltpu.make_async_remote_copy(
                src_ref=o_acc, dst_ref=o_rcv,
                send_sem=send_sems.at[0], recv_sem=recv_sems.at[0],
                device_id=peer, device_id_type=pl.DeviceIdType.MESH,
            )
            rdma_ml = pltpu.make_async_remote_copy(
                src_ref=ml_acc, dst_ref=ml_rcv,
                send_sem=send_sems.at[1], recv_sem=recv_sems.at[1],
                device_id=peer, device_id_type=pl.DeviceIdType.MESH,
            )
            rdma_o.start()
            rdma_ml.start()
            rdma_o.wait()
            rdma_ml.wait()

            ma, la = ml_acc[0], ml_acc[1]
            mb, lb = ml_rcv[0], ml_rcv[1]
            mm = jnp.maximum(ma, mb)
            ea = jnp.exp(ma - mm)
            eb = jnp.exp(mb - mm)
            lt = la * ea + lb * eb
            ot = o_acc[...] * ea + o_rcv[...] * eb
            inv = 1.0 / lt
            for hh in range(H):
                out_ref[:, hh * D:(hh + 1) * D] = ot[hh] * inv[hh]

    out2 = pl.pallas_call(
        body,
        grid=(H,),
        in_specs=[
            pl.BlockSpec((B, D), lambda h: (0, h)),
            pl.BlockSpec((KEYS, D), lambda h: (0, h)),
            pl.BlockSpec((KEYS, D), lambda h: (0, h)),
            pl.BlockSpec((B, KEYS), lambda h: (0, 0)),
        ],
        out_specs=pl.BlockSpec((B, H * D), lambda h: (0, 0)),
        out_shape=jax.ShapeDtypeStruct((B, H * D), jnp.float32),
        scratch_shapes=[
            pltpu.VMEM((H, B, D), jnp.float32),
            pltpu.VMEM((2, H, B, 1), jnp.float32),
            pltpu.VMEM((H, B, D), jnp.float32),
            pltpu.VMEM((2, H, B, 1), jnp.float32),
            pltpu.SemaphoreType.DMA((2,)),
            pltpu.SemaphoreType.DMA((2,)),
        ],
        compiler_params=pltpu.CompilerParams(
            dimension_semantics=("arbitrary",),
        ),
    )(q2, k2, v2, logw_keys)
    return out2.reshape(B, 1, H, D)


# device time: 158683 ns/iter; 1.0336x vs baseline; 1.0336x over previous
import jax
import jax.numpy as jnp
from jax import lax
from jax.experimental import pallas as pl
from jax.experimental.pallas import tpu as pltpu

B = 32
NB = 256
BS = 32
H = 16
D = 128
P_GLOBAL = 512
P_LOCAL = 256
KEYS = P_LOCAL * BS
NEG = -1e9
SCALE = D ** -0.5


def kernel(Q, K, V, bt, lens):
    q2 = Q.reshape(B, H * D)
    k2 = K.reshape(KEYS, H * D)
    v2 = V.reshape(KEYS, H * D)
    lens2 = lens.reshape(B, 1)

    def body(q_ref, k_ref, v_ref, bt_ref, lens_ref, out_ref,
             lw_ref, o_acc, ml_acc, o_rcv, ml_rcv, send_sems, recv_sems):
        h = pl.program_id(0)

        @pl.when(h == 0)
        def _():
            my_x = lax.axis_index("x")
            page_ids = (
                lax.broadcasted_iota(jnp.int32, (1, 1, P_LOCAL), 2)
                + my_x * P_LOCAL
            )
            slot_ids = lax.broadcasted_iota(jnp.int32, (1, NB, 1), 1)
            hits = (bt_ref[...][:, :, None] == page_ids) & (
                slot_ids < lens_ref[...][:, :, None]
            )
            W = jnp.sum(hits.astype(jnp.float32), axis=1)
            logw = jnp.where(W > 0, jnp.log(jnp.maximum(W, 1e-37)), NEG)
            erow = lax.broadcasted_iota(jnp.int32, (P_LOCAL, KEYS), 0)
            ecol = lax.broadcasted_iota(jnp.int32, (P_LOCAL, KEYS), 1)
            E = (ecol // BS == erow).astype(jnp.float32)
            lw_ref[...] = lax.dot_general(
                logw, E, (((1,), (0,)), ((), ())),
                preferred_element_type=jnp.float32,
            )
        q = q_ref[...].astype(jnp.bfloat16)
        k = k_ref[...].astype(jnp.bfloat16)
        v = v_ref[...].astype(jnp.bfloat16)
        s = lax.dot_general(
            q, k, (((1,), (1,)), ((), ())),
            preferred_element_type=jnp.float32,
        )
        s = s * SCALE + lw_ref[...]
        m = jnp.max(s, axis=1, keepdims=True)
        p = jnp.where(m > -1e8, jnp.exp(s - m), 0.0)
        l = jnp.sum(p, axis=1, keepdims=True)
        o = lax.dot_general(
            p.astype(jnp.bfloat16), v, (((1,), (0,)), ((), ())),
            preferred_element_type=jnp.float32,
        )
        o_acc[h] = o
        ml_acc[0, h] = m
        ml_acc[1, h] = l

        @pl.when(h == H - 1)
        def _():
            peer = (1 - lax.axis_index("x"), lax.axis_index("y"))
            rdma_o = pltpu.make_async_remote_copy(
                src_ref=o_acc, dst_ref=o_rcv,
                send_sem=send_sems.at[0], recv_sem=recv_sems.at[0],
                device_id=peer, device_id_type=pl.DeviceIdType.MESH,
            )
            rdma_ml = pltpu.make_async_remote_copy(
                src_ref=ml_acc, dst_ref=ml_rcv,
                send_sem=send_sems.at[1], recv_sem=recv_sems.at[1],
                device_id=peer, device_id_type=pl.DeviceIdType.MESH,
            )
            rdma_o.start()
            rdma_ml.start()
            rdma_o.wait()
            rdma_ml.wait()

            ma, la = ml_acc[0], ml_acc[1]
            mb, lb = ml_rcv[0], ml_rcv[1]
            mm = jnp.maximum(ma, mb)
            ea = jnp.exp(ma - mm)
            eb = jnp.exp(mb - mm)
            lt = la * ea + lb * eb
            ot = o_acc[...] * ea + o_rcv[...] * eb
            inv = 1.0 / lt
            for hh in range(H):
                out_ref[:, hh * D:(hh + 1) * D] = ot[hh] * inv[hh]

    out2 = pl.pallas_call(
        body,
        grid=(H,),
        in_specs=[
            pl.BlockSpec((B, D), lambda h: (0, h)),
            pl.BlockSpec((KEYS, D), lambda h: (0, h)),
            pl.BlockSpec((KEYS, D), lambda h: (0, h)),
            pl.BlockSpec((B, NB), lambda h: (0, 0)),
            pl.BlockSpec((B, 1), lambda h: (0, 0)),
        ],
        out_specs=pl.BlockSpec((B, H * D), lambda h: (0, 0)),
        out_shape=jax.ShapeDtypeStruct((B, H * D), jnp.float32),
        scratch_shapes=[
            pltpu.VMEM((B, KEYS), jnp.float32),
            pltpu.VMEM((H, B, D), jnp.float32),
            pltpu.VMEM((2, H, B, 1), jnp.float32),
            pltpu.VMEM((H, B, D), jnp.float32),
            pltpu.VMEM((2, H, B, 1), jnp.float32),
            pltpu.SemaphoreType.DMA((2,)),
            pltpu.SemaphoreType.DMA((2,)),
        ],
        compiler_params=pltpu.CompilerParams(
            dimension_semantics=("arbitrary",),
        ),
    )(q2, k2, v2, bt, lens2)
    return out2.reshape(B, 1, H, D)


# device time: 65460 ns/iter; 2.5056x vs baseline; 2.4241x over previous
import jax
import jax.numpy as jnp
from jax import lax
from jax.experimental import pallas as pl
from jax.experimental.pallas import tpu as pltpu

B = 32
NB = 256
BS = 32
H = 16
D = 128
P_GLOBAL = 512
P_LOCAL = 256
KEYS = P_LOCAL * BS
NEG = -1e9
SCALE = D ** -0.5


def kernel(Q, K, V, bt, lens):
    lens2 = lens.reshape(B, 1)

    def body(q_ref, k_hbm, v_hbm, bt_ref, lens_ref, out_ref,
             lw_ref, o_acc, ml_acc, o_rcv, ml_rcv,
             kbuf, vbuf, kv_sems, send_sems, recv_sems):
        h = pl.program_id(0)
        slot = h % 2
        nxt = (h + 1) % 2

        @pl.when(h == 0)
        def _():
            pltpu.make_async_copy(
                k_hbm.at[:, :, 0, :], kbuf.at[0], kv_sems.at[0, 0]).start()
            pltpu.make_async_copy(
                v_hbm.at[:, :, 0, :], vbuf.at[0], kv_sems.at[1, 0]).start()

            my_x = lax.axis_index("x")
            page_ids = (
                lax.broadcasted_iota(jnp.int32, (1, 1, P_LOCAL), 2)
                + my_x * P_LOCAL
            )
            slot_ids = lax.broadcasted_iota(jnp.int32, (1, NB, 1), 1)
            hits = (bt_ref[...][:, :, None] == page_ids) & (
                slot_ids < lens_ref[...][:, :, None]
            )
            W = jnp.sum(hits.astype(jnp.float32), axis=1)
            logw = jnp.where(W > 0, jnp.log(jnp.maximum(W, 1e-37)), NEG)
            erow = lax.broadcasted_iota(jnp.int32, (P_LOCAL, KEYS), 0)
            ecol = lax.broadcasted_iota(jnp.int32, (P_LOCAL, KEYS), 1)
            E = (ecol // BS == erow).astype(jnp.float32)
            lw_ref[...] = lax.dot_general(
                logw, E, (((1,), (0,)), ((), ())),
                preferred_element_type=jnp.float32,
            )

        @pl.when(h + 1 < H)
        def _():
            pltpu.make_async_copy(
                k_hbm.at[:, :, h + 1, :], kbuf.at[nxt],
                kv_sems.at[0, nxt]).start()
            pltpu.make_async_copy(
                v_hbm.at[:, :, h + 1, :], vbuf.at[nxt],
                kv_sems.at[1, nxt]).start()

        pltpu.make_async_copy(
            k_hbm.at[:, :, h, :], kbuf.at[slot], kv_sems.at[0, slot]).wait()
        pltpu.make_async_copy(
            v_hbm.at[:, :, h, :], vbuf.at[slot], kv_sems.at[1, slot]).wait()

        q = q_ref[:, 0, h, :].astype(jnp.bfloat16)
        k = kbuf[slot].reshape(KEYS, D).astype(jnp.bfloat16)
        v = vbuf[slot].reshape(KEYS, D).astype(jnp.bfloat16)
        s = lax.dot_general(
            q, k, (((1,), (1,)), ((), ())),
            preferred_element_type=jnp.float32,
        )
        s = s * SCALE + lw_ref[...]
        m = jnp.max(s, axis=1, keepdims=True)
        p = jnp.where(m > -1e8, jnp.exp(s - m), 0.0)
        l = jnp.sum(p, axis=1, keepdims=True)
        o = lax.dot_general(
            p.astype(jnp.bfloat16), v, (((1,), (0,)), ((), ())),
            preferred_element_type=jnp.float32,
        )
        o_acc[h] = o
        ml_acc[0, h] = m
        ml_acc[1, h] = l

        @pl.when(h == H - 1)
        def _():
            peer = (1 - lax.axis_index("x"), lax.axis_index("y"))
            rdma_o = pltpu.make_async_remote_copy(
                src_ref=o_acc, dst_ref=o_rcv,
                send_sem=send_sems.at[0], recv_sem=recv_sems.at[0],
                device_id=peer, device_id_type=pl.DeviceIdType.MESH,
            )
            rdma_ml = pltpu.make_async_remote_copy(
                src_ref=ml_acc, dst_ref=ml_rcv,
                send_sem=send_sems.at[1], recv_sem=recv_sems.at[1],
                device_id=peer, device_id_type=pl.DeviceIdType.MESH,
            )
            rdma_o.start()
            rdma_ml.start()
            rdma_o.wait()
            rdma_ml.wait()

            ma, la = ml_acc[0], ml_acc[1]
            mb, lb = ml_rcv[0], ml_rcv[1]
            mm = jnp.maximum(ma, mb)
            ea = jnp.exp(ma - mm)
            eb = jnp.exp(mb - mm)
            lt = la * ea + lb * eb
            ot = o_acc[...] * ea + o_rcv[...] * eb
            inv = 1.0 / lt
            for hh in range(H):
                out_ref[:, hh * D:(hh + 1) * D] = ot[hh] * inv[hh]

    out2 = pl.pallas_call(
        body,
        grid=(H,),
        in_specs=[
            pl.BlockSpec((B, 1, H, D), lambda h: (0, 0, 0, 0)),
            pl.BlockSpec(memory_space=pltpu.MemorySpace.HBM),
            pl.BlockSpec(memory_space=pltpu.MemorySpace.HBM),
            pl.BlockSpec((B, NB), lambda h: (0, 0)),
            pl.BlockSpec((B, 1), lambda h: (0, 0)),
        ],
        out_specs=pl.BlockSpec((B, H * D), lambda h: (0, 0)),
        out_shape=jax.ShapeDtypeStruct((B, H * D), jnp.float32),
        scratch_shapes=[
            pltpu.VMEM((B, KEYS), jnp.float32),
            pltpu.VMEM((H, B, D), jnp.float32),
            pltpu.VMEM((2, H, B, 1), jnp.float32),
            pltpu.VMEM((H, B, D), jnp.float32),
            pltpu.VMEM((2, H, B, 1), jnp.float32),
            pltpu.VMEM((2, P_LOCAL, BS, D), jnp.float32),
            pltpu.VMEM((2, P_LOCAL, BS, D), jnp.float32),
            pltpu.SemaphoreType.DMA((2, 2)),
            pltpu.SemaphoreType.DMA((2,)),
            pltpu.SemaphoreType.DMA((2,)),
        ],
        compiler_params=pltpu.CompilerParams(
            dimension_semantics=("arbitrary",),
        ),
    )(Q, K, V, bt, lens2)
    return out2.reshape(B, 1, H, D)


# device time: 44487 ns/iter; 3.6869x vs baseline; 1.4714x over previous
import jax
import jax.numpy as jnp
from jax import lax
from jax.experimental import pallas as pl
from jax.experimental.pallas import tpu as pltpu

B = 32
NB = 256
BS = 32
H = 16
HL = H // 2
D = 128
P_GLOBAL = 512
P_LOCAL = 256
KEYS = P_LOCAL * BS
NEG = -1e9
SCALE = D ** -0.5


def kernel(Q, K, V, bt, lens):
    lens2 = lens.reshape(B, 1)

    def body(q_ref, k_hbm, v_hbm, bt_ref, lens_ref, out_ref,
             lw_ref, o_acc, ml_acc, o_rcv, ml_rcv,
             kbuf, vbuf, kv_sems, send_sems, recv_sems):
        h = pl.program_id(0)
        my_y = lax.axis_index("y")
        head = my_y * HL + h
        slot = h % 2
        nxt = (h + 1) % 2

        @pl.when(h == 0)
        def _():
            pltpu.make_async_copy(
                k_hbm.at[:, :, head, :], kbuf.at[0], kv_sems.at[0, 0]).start()
            pltpu.make_async_copy(
                v_hbm.at[:, :, head, :], vbuf.at[0], kv_sems.at[1, 0]).start()

            my_x = lax.axis_index("x")
            page_ids = (
                lax.broadcasted_iota(jnp.int32, (1, 1, P_LOCAL), 2)
                + my_x * P_LOCAL
            )
            slot_ids = lax.broadcasted_iota(jnp.int32, (1, NB, 1), 1)
            hits = (bt_ref[...][:, :, None] == page_ids) & (
                slot_ids < lens_ref[...][:, :, None]
            )
            W = jnp.sum(hits.astype(jnp.float32), axis=1)
            logw = jnp.where(W > 0, jnp.log(jnp.maximum(W, 1e-37)), NEG)
            erow = lax.broadcasted_iota(jnp.int32, (P_LOCAL, KEYS), 0)
            ecol = lax.broadcasted_iota(jnp.int32, (P_LOCAL, KEYS), 1)
            E = (ecol // BS == erow).astype(jnp.float32)
            lw_ref[...] = lax.dot_general(
                logw, E, (((1,), (0,)), ((), ())),
                preferred_element_type=jnp.float32,
            )

        @pl.when(h + 1 < HL)
        def _():
            pltpu.make_async_copy(
                k_hbm.at[:, :, head + 1, :], kbuf.at[nxt],
                kv_sems.at[0, nxt]).start()
            pltpu.make_async_copy(
                v_hbm.at[:, :, head + 1, :], vbuf.at[nxt],
                kv_sems.at[1, nxt]).start()

        pltpu.make_async_copy(
            k_hbm.at[:, :, head, :], kbuf.at[slot], kv_sems.at[0, slot]).wait()
        pltpu.make_async_copy(
            v_hbm.at[:, :, head, :], vbuf.at[slot], kv_sems.at[1, slot]).wait()

        q = q_ref[:, 0, head, :].astype(jnp.bfloat16)
        k = kbuf[slot].reshape(KEYS, D).astype(jnp.bfloat16)
        v = vbuf[slot].reshape(KEYS, D).astype(jnp.bfloat16)
        s = lax.dot_general(
            q, k, (((1,), (1,)), ((), ())),
            preferred_element_type=jnp.float32,
        )
        s = s * SCALE + lw_ref[...]
        m = jnp.max(s, axis=1, keepdims=True)
        p = jnp.where(m > -1e8, jnp.exp(s - m), 0.0)
        l = jnp.sum(p, axis=1, keepdims=True)
        o = lax.dot_general(
            p.astype(jnp.bfloat16), v, (((1,), (0,)), ((), ())),
            preferred_element_type=jnp.float32,
        )
        o_acc[h] = o
        ml_acc[0, h] = m
        ml_acc[1, h] = l

        @pl.when(h == HL - 1)
        def _():
            my_x = lax.axis_index("x")
            peer_x = (1 - my_x, my_y)
            rdma_o = pltpu.make_async_remote_copy(
                src_ref=o_acc, dst_ref=o_rcv,
                send_sem=send_sems.at[0], recv_sem=recv_sems.at[0],
                device_id=peer_x, device_id_type=pl.DeviceIdType.MESH,
            )
            rdma_ml = pltpu.make_async_remote_copy(
                src_ref=ml_acc, dst_ref=ml_rcv,
                send_sem=send_sems.at[1], recv_sem=recv_sems.at[1],
                device_id=peer_x, device_id_type=pl.DeviceIdType.MESH,
            )
            rdma_o.start()
            rdma_ml.start()
            rdma_o.wait()
            rdma_ml.wait()

            ma, la = ml_acc[0], ml_acc[1]
            mb, lb = ml_rcv[0], ml_rcv[1]
            mm = jnp.maximum(ma, mb)
            ea = jnp.exp(ma - mm)
            eb = jnp.exp(mb - mm)
            lt = la * ea + lb * eb
            ot = o_acc[...] * ea + o_rcv[...] * eb
            inv = 1.0 / lt
            base = my_y * (HL * D)
            for hh in range(HL):
                out_ref[:, pl.ds(base + hh * D, D)] = ot[hh] * inv[hh]

            rdma_y = pltpu.make_async_remote_copy(
                src_ref=out_ref.at[:, pl.ds(base, HL * D)],
                dst_ref=out_ref.at[:, pl.ds(base, HL * D)],
                send_sem=send_sems.at[2], recv_sem=recv_sems.at[2],
                device_id=(my_x, 1 - my_y),
                device_id_type=pl.DeviceIdType.MESH,
            )
            rdma_y.start()
            rdma_y.wait()

    out2 = pl.pallas_call(
        body,
        grid=(HL,),
        in_specs=[
            pl.BlockSpec((B, 1, H, D), lambda h: (0, 0, 0, 0)),
            pl.BlockSpec(memory_space=pltpu.MemorySpace.HBM),
            pl.BlockSpec(memory_space=pltpu.MemorySpace.HBM),
            pl.BlockSpec((B, NB), lambda h: (0, 0)),
            pl.BlockSpec((B, 1), lambda h: (0, 0)),
        ],
        out_specs=pl.BlockSpec((B, H * D), lambda h: (0, 0)),
        out_shape=jax.ShapeDtypeStruct((B, H * D), jnp.float32),
        scratch_shapes=[
            pltpu.VMEM((B, KEYS), jnp.float32),
            pltpu.VMEM((HL, B, D), jnp.float32),
            pltpu.VMEM((2, HL, B, 1), jnp.float32),
            pltpu.VMEM((HL, B, D), jnp.float32),
            pltpu.VMEM((2, HL, B, 1), jnp.float32),
            pltpu.VMEM((2, P_LOCAL, BS, D), jnp.float32),
            pltpu.VMEM((2, P_LOCAL, BS, D), jnp.float32),
            pltpu.SemaphoreType.DMA((2, 2)),
            pltpu.SemaphoreType.DMA((3,)),
            pltpu.SemaphoreType.DMA((3,)),
        ],
        compiler_params=pltpu.CompilerParams(
            dimension_semantics=("arbitrary",),
        ),
    )(Q, K, V, bt, lens2)
    return out2.reshape(B, 1, H, D)


# device time: 38798 ns/iter; 4.2275x vs baseline; 1.1466x over previous
import jax
import jax.numpy as jnp
from jax import lax
from jax.experimental import pallas as pl
from jax.experimental.pallas import tpu as pltpu

B = 32
NB = 256
BS = 32
H = 16
HL = H // 2
D = 128
P_GLOBAL = 512
P_LOCAL = 256
KEYS = P_LOCAL * BS
NEG = -1e9
SCALE = D ** -0.5


def kernel(Q, K, V, bt, lens):
    lens2 = lens.reshape(B, 1)

    def body(q_ref, k_hbm, v_hbm, bt_ref, lens_ref, out_ref,
             lw_ref, o_acc, ml_acc, o_rcv, ml_rcv,
             kbuf, vbuf, kv_sems, send_sems, recv_sems):
        h = pl.program_id(0)
        my_y = lax.axis_index("y")
        head = my_y * HL + h
        slot = h % 2
        nxt = (h + 1) % 2

        @pl.when(h == 0)
        def _():
            pltpu.make_async_copy(
                k_hbm.at[:, :, head, :], kbuf.at[0], kv_sems.at[0, 0]).start()
            pltpu.make_async_copy(
                v_hbm.at[:, :, head, :], vbuf.at[0], kv_sems.at[1, 0]).start()

            my_x = lax.axis_index("x")
            barrier_sem = pltpu.get_barrier_semaphore()
            pl.semaphore_signal(
                barrier_sem, inc=1, device_id=(1 - my_x, my_y),
                device_id_type=pl.DeviceIdType.MESH)
            pl.semaphore_signal(
                barrier_sem, inc=1, device_id=(my_x, 1 - my_y),
                device_id_type=pl.DeviceIdType.MESH)
            pl.semaphore_wait(barrier_sem, 2)
            page_ids = (
                lax.broadcasted_iota(jnp.int32, (1, 1, P_LOCAL), 2)
                + my_x * P_LOCAL
            )
            slot_ids = lax.broadcasted_iota(jnp.int32, (1, NB, 1), 1)
            hits = (bt_ref[...][:, :, None] == page_ids) & (
                slot_ids < lens_ref[...][:, :, None]
            )
            W = jnp.sum(hits.astype(jnp.float32), axis=1)
            logw = jnp.where(W > 0, jnp.log(jnp.maximum(W, 1e-37)), NEG)
            erow = lax.broadcasted_iota(jnp.int32, (P_LOCAL, KEYS), 0)
            ecol = lax.broadcasted_iota(jnp.int32, (P_LOCAL, KEYS), 1)
            E = (ecol // BS == erow).astype(jnp.bfloat16)
            lw_ref[...] = lax.dot_general(
                logw.astype(jnp.bfloat16), E, (((1,), (0,)), ((), ())),
                preferred_element_type=jnp.float32,
            )

        @pl.when(h + 1 < HL)
        def _():
            pltpu.make_async_copy(
                k_hbm.at[:, :, head + 1, :], kbuf.at[nxt],
                kv_sems.at[0, nxt]).start()
            pltpu.make_async_copy(
                v_hbm.at[:, :, head + 1, :], vbuf.at[nxt],
                kv_sems.at[1, nxt]).start()

        pltpu.make_async_copy(
            k_hbm.at[:, :, head, :], kbuf.at[slot], kv_sems.at[0, slot]).wait()
        pltpu.make_async_copy(
            v_hbm.at[:, :, head, :], vbuf.at[slot], kv_sems.at[1, slot]).wait()

        q = q_ref[:, 0, head, :].astype(jnp.bfloat16)
        k = kbuf[slot].reshape(KEYS, D).astype(jnp.bfloat16)
        v = vbuf[slot].reshape(KEYS, D).astype(jnp.bfloat16)
        s = lax.dot_general(
            q, k, (((1,), (1,)), ((), ())),
            preferred_element_type=jnp.float32,
        )
        s = s * SCALE + lw_ref[...]
        m = jnp.max(s, axis=1, keepdims=True)
        p = jnp.where(m > -1e8, jnp.exp(s - m), 0.0)
        l = jnp.sum(p, axis=1, keepdims=True)
        o = lax.dot_general(
            p.astype(jnp.bfloat16), v, (((1,), (0,)), ((), ())),
            preferred_element_type=jnp.float32,
        )
        o_acc[h] = o
        ml_acc[0, h] = m
        ml_acc[1, h] = l

        @pl.when(h == HL - 1)
        def _():
            my_x = lax.axis_index("x")
            peer_x = (1 - my_x, my_y)
            rdma_o = pltpu.make_async_remote_copy(
                src_ref=o_acc, dst_ref=o_rcv,
                send_sem=send_sems.at[0], recv_sem=recv_sems.at[0],
                device_id=peer_x, device_id_type=pl.DeviceIdType.MESH,
            )
            rdma_ml = pltpu.make_async_remote_copy(
                src_ref=ml_acc, dst_ref=ml_rcv,
                send_sem=send_sems.at[1], recv_sem=recv_sems.at[1],
                device_id=peer_x, device_id_type=pl.DeviceIdType.MESH,
            )
            rdma_o.start()
            rdma_ml.start()
            rdma_o.wait()
            rdma_ml.wait()

            ma, la = ml_acc[0], ml_acc[1]
            mb, lb = ml_rcv[0], ml_rcv[1]
            mm = jnp.maximum(ma, mb)
            ea = jnp.exp(ma - mm)
            eb = jnp.exp(mb - mm)
            lt = la * ea + lb * eb
            ot = o_acc[...] * ea + o_rcv[...] * eb
            inv = 1.0 / lt
            base = my_y * (HL * D)
            for hh in range(HL):
                out_ref[:, pl.ds(base + hh * D, D)] = ot[hh] * inv[hh]

            rdma_y = pltpu.make_async_remote_copy(
                src_ref=out_ref.at[:, pl.ds(base, HL * D)],
                dst_ref=out_ref.at[:, pl.ds(base, HL * D)],
                send_sem=send_sems.at[2], recv_sem=recv_sems.at[2],
                device_id=(my_x, 1 - my_y),
                device_id_type=pl.DeviceIdType.MESH,
            )
            rdma_y.start()
            rdma_y.wait()

    out2 = pl.pallas_call(
        body,
        grid=(HL,),
        in_specs=[
            pl.BlockSpec((B, 1, H, D), lambda h: (0, 0, 0, 0)),
            pl.BlockSpec(memory_space=pltpu.MemorySpace.HBM),
            pl.BlockSpec(memory_space=pltpu.MemorySpace.HBM),
            pl.BlockSpec((B, NB), lambda h: (0, 0)),
            pl.BlockSpec((B, 1), lambda h: (0, 0)),
        ],
        out_specs=pl.BlockSpec((B, H * D), lambda h: (0, 0)),
        out_shape=jax.ShapeDtypeStruct((B, H * D), jnp.float32),
        scratch_shapes=[
            pltpu.VMEM((B, KEYS), jnp.float32),
            pltpu.VMEM((HL, B, D), jnp.float32),
            pltpu.VMEM((2, HL, B, 1), jnp.float32),
            pltpu.VMEM((HL, B, D), jnp.float32),
            pltpu.VMEM((2, HL, B, 1), jnp.float32),
            pltpu.VMEM((2, P_LOCAL, BS, D), jnp.float32),
            pltpu.VMEM((2, P_LOCAL, BS, D), jnp.float32),
            pltpu.SemaphoreType.DMA((2, 2)),
            pltpu.SemaphoreType.DMA((3,)),
            pltpu.SemaphoreType.DMA((3,)),
        ],
        compiler_params=pltpu.CompilerParams(
            dimension_semantics=("arbitrary",),
            collective_id=0,
        ),
    )(Q, K, V, bt, lens2)
    return out2.reshape(B, 1, H, D)
